# baseline (speedup 1.0000x reference)
import jax
import jax.numpy as jnp
from jax import lax
from jax.experimental import pallas as pl
from jax.experimental.pallas import tpu as pltpu
from jax.experimental.pallas import tpu_sc as plsc

_K = 1024
_C = 768
_B = 16384
_NC = 2
_NS = 16
_NW = _NC * _NS
_RPW = _B // _NW
_R = 32
_NCHUNK = _RPW // _R
_KPT = _K // _NS
_CW = 128
_BLK = 512


def _zero_tables(sid, fbuf, cbuf, sums_sh, cnt_sh):
    zero16 = jnp.zeros((16,), jnp.float32)

    def _zr(i, carry):
        for j in range(_C // 16):
            fbuf[i, pl.ds(j * 16, 16)] = zero16
        for j in range(_CW // 16):
            cbuf[i, pl.ds(j * 16, 16)] = zero16
        return carry
    lax.fori_loop(0, _R, _zr, 0)

    ksl = pl.ds(sid * _KPT, _KPT)
    for h in range(_KPT // _R):
        hsl = pl.ds(sid * _KPT + h * _R, _R)
        pltpu.sync_copy(fbuf, sums_sh.at[hsl])
        pltpu.sync_copy(cbuf, cnt_sh.at[hsl])
    plsc.subcore_barrier()
    return ksl


def _pipeline(wid, sums_sh, cnt_sh, fb, cb, lb, isem, ssem, feat, lbl):
    def start_in(g):
        s = g & 1
        base = wid * _RPW + g * _R
        return [
            pltpu.async_copy(lbl.at[pl.ds(base, _R)], lb[s].at[0], isem[s]),
            pltpu.async_copy(feat.at[pl.ds(base, _R), :], fb[s], isem[s]),
        ]

    def start_scatter(g):
        s = g & 1
        return [
            pltpu.async_copy(fb[s], sums_sh.at[lb[s].at[0]], ssem[s],
                             add=True),
            pltpu.async_copy(cb[0], cnt_sh.at[lb[s].at[0]], ssem[s],
                             add=True),
        ]

    ind = {0: start_in(0)}
    scd = {}
    for g in range(_NCHUNK):
        if g + 1 < _NCHUNK:
            if g >= 1:
                for d in scd[g - 1]:
                    d.wait()
            ind[g + 1] = start_in(g + 1)
        for d in ind[g]:
            d.wait()
        scd[g] = start_scatter(g)
    for d in scd[_NCHUNK - 2]:
        d.wait()
    for d in scd[_NCHUNK - 1]:
        d.wait()


def _src_body(feat, lbl, sums_o, cnt_o, sums_sh, cnt_sh,
              fbuf0, fbuf1, cbuf0, cbuf1, lbl0, lbl1,
              isem0, isem1, ssem0, ssem1):
    cid = lax.axis_index("c")
    sid = lax.axis_index("s")
    ksl = _zero_tables(sid, fbuf0, cbuf0, sums_sh, cnt_sh)

    iota16 = lax.iota(jnp.int32, 16)
    one0 = (1 - jnp.minimum(iota16, 1)).astype(jnp.float32)
    zero16 = jnp.zeros((16,), jnp.float32)
    def _fill(i, carry):
        cbuf0[i, pl.ds(0, 16)] = one0
        for j in range(1, _CW // 16):
            cbuf0[i, pl.ds(j * 16, 16)] = zero16
        return carry
    lax.fori_loop(0, _R, _fill, 0)

    wid = sid * _NC + cid
    _pipeline(wid, sums_sh, cnt_sh, (fbuf0, fbuf1), (cbuf0, cbuf1),
              (lbl0, lbl1), (isem0, isem1), (ssem0, ssem1), feat, lbl)

    plsc.subcore_barrier()
    pltpu.sync_copy(sums_sh.at[ksl], sums_o.at[cid, ksl])
    pltpu.sync_copy(cnt_sh.at[ksl], cnt_o.at[cid, ksl])


def _segment_src(src_features, src_labels):
    mesh = plsc.VectorSubcoreMesh(core_axis_name="c", subcore_axis_name="s",
                                  num_cores=_NC, num_subcores=_NS)
    f32 = jnp.float32
    out_type = (
        jax.ShapeDtypeStruct((_NC, _K, _C), f32),
        jax.ShapeDtypeStruct((_NC, _K, _CW), f32),
    )
    scratch = [
        pltpu.VMEM_SHARED((_K, _C), f32),
        pltpu.VMEM_SHARED((_K, _CW), f32),
        pltpu.VMEM((_R, _C), f32),
        pltpu.VMEM((_R, _C), f32),
        pltpu.VMEM((_R, _CW), f32),
        pltpu.VMEM((_R, _CW), f32),
        pltpu.VMEM((1, _R), jnp.int32),
        pltpu.VMEM((1, _R), jnp.int32),
        pltpu.SemaphoreType.DMA,
        pltpu.SemaphoreType.DMA,
        pltpu.SemaphoreType.DMA,
        pltpu.SemaphoreType.DMA,
    ]
    cp = pltpu.CompilerParams(use_tc_tiling_on_sc=False, skip_device_barrier=True)
    return pl.kernel(
        _src_body, out_type=out_type, mesh=mesh, scratch_types=scratch,
        compiler_params=cp,
    )(src_features, src_labels)


def _trgsum_body(lbl_ref, f_ref, c_ref, sums_ref, cnt_ref):
    @pl.when(pl.program_id(0) == 0)
    def _():
        sums_ref[...] = jnp.zeros_like(sums_ref)
        cnt_ref[...] = jnp.zeros_like(cnt_ref)

    k_iota = lax.broadcasted_iota(jnp.int32, (_BLK, _K), 1)
    onehot = (lbl_ref[...] == k_iota).astype(jnp.bfloat16)
    conf = c_ref[...]
    dn0 = (((0,), (0,)), ((), ()))
    wf = f_ref[...] * conf
    hi = wf.astype(jnp.bfloat16)
    lo = (wf - hi.astype(jnp.float32)).astype(jnp.bfloat16)
    sums_ref[...] += (
        lax.dot_general(onehot, hi, dimension_numbers=dn0,
                        preferred_element_type=jnp.float32)
        + lax.dot_general(onehot, lo, dimension_numbers=dn0,
                          preferred_element_type=jnp.float32))
    chi = conf.astype(jnp.bfloat16)
    clo = (conf - chi.astype(jnp.float32)).astype(jnp.bfloat16)
    ones = jnp.ones((_BLK, 1), jnp.bfloat16)
    cw_hi = jnp.concatenate([chi, ones], axis=1)
    cw_lo = jnp.concatenate([clo, jnp.zeros((_BLK, 1), jnp.bfloat16)],
                            axis=1)
    cnt_ref[...] += (
        lax.dot_general(onehot, cw_hi, dimension_numbers=dn0,
                        preferred_element_type=jnp.float32)
        + lax.dot_general(onehot, cw_lo, dimension_numbers=dn0,
                          preferred_element_type=jnp.float32))


def _trg_matmul(trg_labels_col, trg_features, conf_col):
    return pl.pallas_call(
        _trgsum_body,
        grid=(_B // _BLK,),
        in_specs=[
            pl.BlockSpec((_BLK, 1), lambda i: (i, 0)),
            pl.BlockSpec((_BLK, _C), lambda i: (i, 0)),
            pl.BlockSpec((_BLK, 1), lambda i: (i, 0)),
        ],
        out_specs=[
            pl.BlockSpec((_K, _C), lambda i: (0, 0)),
            pl.BlockSpec((_K, 2), lambda i: (0, 0)),
        ],
        out_shape=[
            jax.ShapeDtypeStruct((_K, _C), jnp.float32),
            jax.ShapeDtypeStruct((_K, 2), jnp.float32),
        ],
    )(trg_labels_col, trg_features, conf_col)


def _finish_body(ss_ref, sc_ref, ts_ref, tc_ref, o_ref):
    s_sum = ss_ref[0] + ss_ref[1]
    s_cnt = sc_ref[0, :, 0:1] + sc_ref[1, :, 0:1]
    t_wsum = ts_ref[...]
    t_wcnt = tc_ref[:, 0:1]
    t_cnt = tc_ref[:, 1:2]

    s_present = (s_cnt > 0).astype(jnp.float32)
    t_present = (t_cnt > 0).astype(jnp.float32)

    s_ctr = (1.0 - 0.9) * (s_sum / jnp.maximum(s_cnt, 1.0)) * s_present
    t_ctr = (1.0 - 0.9) * (t_wsum / jnp.maximum(t_wcnt, 1e-12)) * t_present

    s_n = jnp.sqrt(jnp.sum(s_ctr * s_ctr, axis=1, keepdims=True))
    t_n = jnp.sqrt(jnp.sum(t_ctr * t_ctr, axis=1, keepdims=True))
    validf = (s_present * t_present
              * (s_n > 1e-6).astype(jnp.float32)
              * (t_n > 1e-6).astype(jnp.float32))
    nv = jnp.sum(validf)

    d = s_ctr - t_ctr
    sq = jnp.sum(d * d * validf)
    denom = jnp.maximum(nv * float(_C), 1.0)
    o_ref[0, 0] = jnp.minimum(nv, 1.0) * (sq / denom)


def _finish(s_sums, s_cnt, t_sums, t_cnt):
    return pl.pallas_call(
        _finish_body,
        out_shape=jax.ShapeDtypeStruct((1, 1), jnp.float32),
        out_specs=pl.BlockSpec(memory_space=pltpu.SMEM),
    )(s_sums, s_cnt, t_sums, t_cnt)


@jax.jit
def kernel(src_features, src_labels, trg_features, trg_pseudo_labels,
           confidence):
    src_labels = src_labels.astype(jnp.int32)
    trg_pseudo_labels = trg_pseudo_labels.astype(jnp.int32)
    conf = confidence.astype(jnp.float32)

    s_sums, s_cnt = _segment_src(src_features, src_labels)
    t_sums, t_cnt = _trg_matmul(trg_pseudo_labels.reshape(_B, 1),
                                trg_features, conf.reshape(_B, 1))
    loss = _finish(s_sums, s_cnt, t_sums, t_cnt)
    return loss[0, 0]


# speedup vs baseline: 1.2629x; 1.2629x over previous
import jax
import jax.numpy as jnp
from jax import lax
from jax.experimental import pallas as pl
from jax.experimental.pallas import tpu as pltpu
from jax.experimental.pallas import tpu_sc as plsc

_K = 1024
_C = 768
_B = 16384
_NC = 2
_NS = 16
_NW = _NC * _NS
_RPW = _B // _NW
_R = 32
_NCHUNK = _RPW // _R
_KPT = _K // _NS
_CW = 128
_BLK = 1024


def _zero_tables(sid, fbuf, cbuf, sums_sh, cnt_sh):
    zero16 = jnp.zeros((16,), jnp.float32)

    def _zr(i, carry):
        for j in range(_C // 16):
            fbuf[i, pl.ds(j * 16, 16)] = zero16
        for j in range(_CW // 16):
            cbuf[i, pl.ds(j * 16, 16)] = zero16
        return carry
    lax.fori_loop(0, _R, _zr, 0)

    ksl = pl.ds(sid * _KPT, _KPT)
    for h in range(_KPT // _R):
        hsl = pl.ds(sid * _KPT + h * _R, _R)
        pltpu.sync_copy(fbuf, sums_sh.at[hsl])
        pltpu.sync_copy(cbuf, cnt_sh.at[hsl])
    plsc.subcore_barrier()
    return ksl


def _pipeline(wid, sums_sh, cnt_sh, fb, cb, lb, isem, ssem, feat, lbl):
    def start_in(g):
        s = g & 1
        base = wid * _RPW + g * _R
        return [
            pltpu.async_copy(lbl.at[pl.ds(base, _R)], lb[s].at[0], isem[s]),
            pltpu.async_copy(feat.at[pl.ds(base, _R), :], fb[s], isem[s]),
        ]

    def start_scatter(g):
        s = g & 1
        return [
            pltpu.async_copy(fb[s], sums_sh.at[lb[s].at[0]], ssem[s],
                             add=True),
            pltpu.async_copy(cb[0], cnt_sh.at[lb[s].at[0]], ssem[s],
                             add=True),
        ]

    ind = {0: start_in(0)}
    scd = {}
    for g in range(_NCHUNK):
        if g + 1 < _NCHUNK:
            if g >= 1:
                for d in scd[g - 1]:
                    d.wait()
            ind[g + 1] = start_in(g + 1)
        for d in ind[g]:
            d.wait()
        scd[g] = start_scatter(g)
    for d in scd[_NCHUNK - 2]:
        d.wait()
    for d in scd[_NCHUNK - 1]:
        d.wait()


def _src_body(feat, lbl, sums_o, cnt_o, sums_sh, cnt_sh,
              fbuf0, fbuf1, cbuf0, cbuf1, lbl0, lbl1,
              isem0, isem1, ssem0, ssem1):
    cid = lax.axis_index("c")
    sid = lax.axis_index("s")
    ksl = _zero_tables(sid, fbuf0, cbuf0, sums_sh, cnt_sh)

    iota16 = lax.iota(jnp.int32, 16)
    one0 = (1 - jnp.minimum(iota16, 1)).astype(jnp.float32)
    zero16 = jnp.zeros((16,), jnp.float32)
    def _fill(i, carry):
        cbuf0[i, pl.ds(0, 16)] = one0
        for j in range(1, _CW // 16):
            cbuf0[i, pl.ds(j * 16, 16)] = zero16
        return carry
    lax.fori_loop(0, _R, _fill, 0)

    wid = sid * _NC + cid
    _pipeline(wid, sums_sh, cnt_sh, (fbuf0, fbuf1), (cbuf0, cbuf1),
              (lbl0, lbl1), (isem0, isem1), (ssem0, ssem1), feat, lbl)

    plsc.subcore_barrier()
    pltpu.sync_copy(sums_sh.at[ksl], sums_o.at[cid, ksl])
    pltpu.sync_copy(cnt_sh.at[ksl], cnt_o.at[cid, ksl])


def _segment_src(src_features, src_labels):
    mesh = plsc.VectorSubcoreMesh(core_axis_name="c", subcore_axis_name="s",
                                  num_cores=_NC, num_subcores=_NS)
    f32 = jnp.float32
    out_type = (
        jax.ShapeDtypeStruct((_NC, _K, _C), f32),
        jax.ShapeDtypeStruct((_NC, _K, _CW), f32),
    )
    scratch = [
        pltpu.VMEM_SHARED((_K, _C), f32),
        pltpu.VMEM_SHARED((_K, _CW), f32),
        pltpu.VMEM((_R, _C), f32),
        pltpu.VMEM((_R, _C), f32),
        pltpu.VMEM((_R, _CW), f32),
        pltpu.VMEM((_R, _CW), f32),
        pltpu.VMEM((1, _R), jnp.int32),
        pltpu.VMEM((1, _R), jnp.int32),
        pltpu.SemaphoreType.DMA,
        pltpu.SemaphoreType.DMA,
        pltpu.SemaphoreType.DMA,
        pltpu.SemaphoreType.DMA,
    ]
    cp = pltpu.CompilerParams(use_tc_tiling_on_sc=False, skip_device_barrier=True)
    return pl.kernel(
        _src_body, out_type=out_type, mesh=mesh, scratch_types=scratch,
        compiler_params=cp,
    )(src_features, src_labels)


def _trgsum_body(lbl_ref, f_ref, c_ref, sums_ref, cnt_ref):
    @pl.when(pl.program_id(0) == 0)
    def _():
        sums_ref[...] = jnp.zeros_like(sums_ref)
        cnt_ref[...] = jnp.zeros_like(cnt_ref)

    k_iota = lax.broadcasted_iota(jnp.int32, (_BLK, _K), 1)
    onehot = (lbl_ref[...] == k_iota).astype(jnp.float32)
    conf = c_ref[...]
    dn0 = (((0,), (0,)), ((), ()))
    sums_ref[...] += lax.dot_general(
        onehot, f_ref[...] * conf, dimension_numbers=dn0,
        preferred_element_type=jnp.float32)
    cw = jnp.concatenate([conf, jnp.ones((_BLK, 1), jnp.float32)], axis=1)
    cnt_ref[...] += lax.dot_general(
        onehot, cw, dimension_numbers=dn0,
        preferred_element_type=jnp.float32)


def _trg_matmul(trg_labels_col, trg_features, conf_col):
    return pl.pallas_call(
        _trgsum_body,
        grid=(_B // _BLK,),
        in_specs=[
            pl.BlockSpec((_BLK, 1), lambda i: (i, 0)),
            pl.BlockSpec((_BLK, _C), lambda i: (i, 0)),
            pl.BlockSpec((_BLK, 1), lambda i: (i, 0)),
        ],
        out_specs=[
            pl.BlockSpec((_K, _C), lambda i: (0, 0)),
            pl.BlockSpec((_K, 2), lambda i: (0, 0)),
        ],
        out_shape=[
            jax.ShapeDtypeStruct((_K, _C), jnp.float32),
            jax.ShapeDtypeStruct((_K, 2), jnp.float32),
        ],
    )(trg_labels_col, trg_features, conf_col)


def _finish_body(ss_ref, sc_ref, ts_ref, tc_ref, o_ref):
    s_sum = ss_ref[0] + ss_ref[1]
    s_cnt = sc_ref[0, :, 0:1] + sc_ref[1, :, 0:1]
    t_wsum = ts_ref[...]
    t_wcnt = tc_ref[:, 0:1]
    t_cnt = tc_ref[:, 1:2]

    s_present = (s_cnt > 0).astype(jnp.float32)
    t_present = (t_cnt > 0).astype(jnp.float32)

    s_ctr = (1.0 - 0.9) * (s_sum / jnp.maximum(s_cnt, 1.0)) * s_present
    t_ctr = (1.0 - 0.9) * (t_wsum / jnp.maximum(t_wcnt, 1e-12)) * t_present

    s_n = jnp.sqrt(jnp.sum(s_ctr * s_ctr, axis=1, keepdims=True))
    t_n = jnp.sqrt(jnp.sum(t_ctr * t_ctr, axis=1, keepdims=True))
    validf = (s_present * t_present
              * (s_n > 1e-6).astype(jnp.float32)
              * (t_n > 1e-6).astype(jnp.float32))
    nv = jnp.sum(validf)

    d = s_ctr - t_ctr
    sq = jnp.sum(d * d * validf)
    denom = jnp.maximum(nv * float(_C), 1.0)
    o_ref[0, 0] = jnp.minimum(nv, 1.0) * (sq / denom)


def _finish(s_sums, s_cnt, t_sums, t_cnt):
    return pl.pallas_call(
        _finish_body,
        out_shape=jax.ShapeDtypeStruct((1, 1), jnp.float32),
        out_specs=pl.BlockSpec(memory_space=pltpu.SMEM),
    )(s_sums, s_cnt, t_sums, t_cnt)


@jax.jit
def kernel(src_features, src_labels, trg_features, trg_pseudo_labels,
           confidence):
    src_labels = src_labels.astype(jnp.int32)
    trg_pseudo_labels = trg_pseudo_labels.astype(jnp.int32)
    conf = confidence.astype(jnp.float32)

    s_sums, s_cnt = _segment_src(src_features, src_labels)
    t_sums, t_cnt = _trg_matmul(trg_pseudo_labels.reshape(_B, 1),
                                trg_features, conf.reshape(_B, 1))
    loss = _finish(s_sums, s_cnt, t_sums, t_cnt)
    return loss[0, 0]


# speedup vs baseline: 1.2766x; 1.0108x over previous
import jax
import jax.numpy as jnp
from jax import lax
from jax.experimental import pallas as pl
from jax.experimental.pallas import tpu as pltpu
from jax.experimental.pallas import tpu_sc as plsc

_K = 1024
_C = 768
_B = 16384
_NC = 2
_NS = 16
_NW = _NC * _NS
_RPW = _B // _NW
_R = 32
_NCHUNK = _RPW // _R
_KPT = _K // _NS
_CW = 128
_BLK = 2048


def _zero_tables(sid, fbuf, cbuf, sums_sh, cnt_sh):
    zero16 = jnp.zeros((16,), jnp.float32)

    def _zr(i, carry):
        for j in range(_C // 16):
            fbuf[i, pl.ds(j * 16, 16)] = zero16
        for j in range(_CW // 16):
            cbuf[i, pl.ds(j * 16, 16)] = zero16
        return carry
    lax.fori_loop(0, _R, _zr, 0)

    ksl = pl.ds(sid * _KPT, _KPT)
    for h in range(_KPT // _R):
        hsl = pl.ds(sid * _KPT + h * _R, _R)
        pltpu.sync_copy(fbuf, sums_sh.at[hsl])
        pltpu.sync_copy(cbuf, cnt_sh.at[hsl])
    plsc.subcore_barrier()
    return ksl


def _pipeline(wid, sums_sh, cnt_sh, fb, cb, lb, isem, ssem, feat, lbl):
    def start_in(g):
        s = g & 1
        base = wid * _RPW + g * _R
        return [
            pltpu.async_copy(lbl.at[pl.ds(base, _R)], lb[s].at[0], isem[s]),
            pltpu.async_copy(feat.at[pl.ds(base, _R), :], fb[s], isem[s]),
        ]

    def start_scatter(g):
        s = g & 1
        return [
            pltpu.async_copy(fb[s], sums_sh.at[lb[s].at[0]], ssem[s],
                             add=True),
            pltpu.async_copy(cb[0], cnt_sh.at[lb[s].at[0]], ssem[s],
                             add=True),
        ]

    ind = {0: start_in(0)}
    scd = {}
    for g in range(_NCHUNK):
        if g + 1 < _NCHUNK:
            if g >= 1:
                for d in scd[g - 1]:
                    d.wait()
            ind[g + 1] = start_in(g + 1)
        for d in ind[g]:
            d.wait()
        scd[g] = start_scatter(g)
    for d in scd[_NCHUNK - 2]:
        d.wait()
    for d in scd[_NCHUNK - 1]:
        d.wait()


def _src_body(feat, lbl, sums_o, cnt_o, sums_sh, cnt_sh,
              fbuf0, fbuf1, cbuf0, cbuf1, lbl0, lbl1,
              isem0, isem1, ssem0, ssem1):
    cid = lax.axis_index("c")
    sid = lax.axis_index("s")
    ksl = _zero_tables(sid, fbuf0, cbuf0, sums_sh, cnt_sh)

    iota16 = lax.iota(jnp.int32, 16)
    one0 = (1 - jnp.minimum(iota16, 1)).astype(jnp.float32)
    zero16 = jnp.zeros((16,), jnp.float32)
    def _fill(i, carry):
        cbuf0[i, pl.ds(0, 16)] = one0
        for j in range(1, _CW // 16):
            cbuf0[i, pl.ds(j * 16, 16)] = zero16
        return carry
    lax.fori_loop(0, _R, _fill, 0)

    wid = sid * _NC + cid
    _pipeline(wid, sums_sh, cnt_sh, (fbuf0, fbuf1), (cbuf0, cbuf1),
              (lbl0, lbl1), (isem0, isem1), (ssem0, ssem1), feat, lbl)

    plsc.subcore_barrier()
    pltpu.sync_copy(sums_sh.at[ksl], sums_o.at[cid, ksl])
    pltpu.sync_copy(cnt_sh.at[ksl], cnt_o.at[cid, ksl])


def _segment_src(src_features, src_labels):
    mesh = plsc.VectorSubcoreMesh(core_axis_name="c", subcore_axis_name="s",
                                  num_cores=_NC, num_subcores=_NS)
    f32 = jnp.float32
    out_type = (
        jax.ShapeDtypeStruct((_NC, _K, _C), f32),
        jax.ShapeDtypeStruct((_NC, _K, _CW), f32),
    )
    scratch = [
        pltpu.VMEM_SHARED((_K, _C), f32),
        pltpu.VMEM_SHARED((_K, _CW), f32),
        pltpu.VMEM((_R, _C), f32),
        pltpu.VMEM((_R, _C), f32),
        pltpu.VMEM((_R, _CW), f32),
        pltpu.VMEM((_R, _CW), f32),
        pltpu.VMEM((1, _R), jnp.int32),
        pltpu.VMEM((1, _R), jnp.int32),
        pltpu.SemaphoreType.DMA,
        pltpu.SemaphoreType.DMA,
        pltpu.SemaphoreType.DMA,
        pltpu.SemaphoreType.DMA,
    ]
    cp = pltpu.CompilerParams(use_tc_tiling_on_sc=False, skip_device_barrier=True)
    return pl.kernel(
        _src_body, out_type=out_type, mesh=mesh, scratch_types=scratch,
        compiler_params=cp,
    )(src_features, src_labels)


def _trgsum_body(lbl_ref, f_ref, c_ref, sums_ref, cnt_ref):
    @pl.when(pl.program_id(0) == 0)
    def _():
        sums_ref[...] = jnp.zeros_like(sums_ref)
        cnt_ref[...] = jnp.zeros_like(cnt_ref)

    k_iota = lax.broadcasted_iota(jnp.int32, (_BLK, _K), 1)
    onehot = (lbl_ref[...] == k_iota).astype(jnp.float32)
    conf = c_ref[...]
    dn0 = (((0,), (0,)), ((), ()))
    sums_ref[...] += lax.dot_general(
        onehot, f_ref[...] * conf, dimension_numbers=dn0,
        preferred_element_type=jnp.float32)
    cw = jnp.concatenate([conf, jnp.ones((_BLK, 1), jnp.float32)], axis=1)
    cnt_ref[...] += lax.dot_general(
        onehot, cw, dimension_numbers=dn0,
        preferred_element_type=jnp.float32)


def _trg_matmul(trg_labels_col, trg_features, conf_col):
    return pl.pallas_call(
        _trgsum_body,
        grid=(_B // _BLK,),
        in_specs=[
            pl.BlockSpec((_BLK, 1), lambda i: (i, 0)),
            pl.BlockSpec((_BLK, _C), lambda i: (i, 0)),
            pl.BlockSpec((_BLK, 1), lambda i: (i, 0)),
        ],
        out_specs=[
            pl.BlockSpec((_K, _C), lambda i: (0, 0)),
            pl.BlockSpec((_K, 2), lambda i: (0, 0)),
        ],
        out_shape=[
            jax.ShapeDtypeStruct((_K, _C), jnp.float32),
            jax.ShapeDtypeStruct((_K, 2), jnp.float32),
        ],
    )(trg_labels_col, trg_features, conf_col)


def _finish_body(ss_ref, sc_ref, ts_ref, tc_ref, o_ref):
    s_sum = ss_ref[0] + ss_ref[1]
    s_cnt = sc_ref[0, :, 0:1] + sc_ref[1, :, 0:1]
    t_wsum = ts_ref[...]
    t_wcnt = tc_ref[:, 0:1]
    t_cnt = tc_ref[:, 1:2]

    s_present = (s_cnt > 0).astype(jnp.float32)
    t_present = (t_cnt > 0).astype(jnp.float32)

    s_ctr = (1.0 - 0.9) * (s_sum / jnp.maximum(s_cnt, 1.0)) * s_present
    t_ctr = (1.0 - 0.9) * (t_wsum / jnp.maximum(t_wcnt, 1e-12)) * t_present

    s_n = jnp.sqrt(jnp.sum(s_ctr * s_ctr, axis=1, keepdims=True))
    t_n = jnp.sqrt(jnp.sum(t_ctr * t_ctr, axis=1, keepdims=True))
    validf = (s_present * t_present
              * (s_n > 1e-6).astype(jnp.float32)
              * (t_n > 1e-6).astype(jnp.float32))
    nv = jnp.sum(validf)

    d = s_ctr - t_ctr
    sq = jnp.sum(d * d * validf)
    denom = jnp.maximum(nv * float(_C), 1.0)
    o_ref[0, 0] = jnp.minimum(nv, 1.0) * (sq / denom)


def _finish(s_sums, s_cnt, t_sums, t_cnt):
    return pl.pallas_call(
        _finish_body,
        out_shape=jax.ShapeDtypeStruct((1, 1), jnp.float32),
        out_specs=pl.BlockSpec(memory_space=pltpu.SMEM),
    )(s_sums, s_cnt, t_sums, t_cnt)


@jax.jit
def kernel(src_features, src_labels, trg_features, trg_pseudo_labels,
           confidence):
    src_labels = src_labels.astype(jnp.int32)
    trg_pseudo_labels = trg_pseudo_labels.astype(jnp.int32)
    conf = confidence.astype(jnp.float32)

    s_sums, s_cnt = _segment_src(src_features, src_labels)
    t_sums, t_cnt = _trg_matmul(trg_pseudo_labels.reshape(_B, 1),
                                trg_features, conf.reshape(_B, 1))
    loss = _finish(s_sums, s_cnt, t_sums, t_cnt)
    return loss[0, 0]
